# baseline (speedup 1.0000x reference)
import jax
import jax.numpy as jnp
from jax.experimental import pallas as pl
from jax.experimental.pallas import tpu as pltpu


def kernel(input_seqs, pos_emb):
    B, L = input_seqs.shape
    Lk, E = pos_emb.shape
    KK = 128
    n_dma = B // KK
    WINDOW = 8

    def body(emb_ref, out_ref, buf, sem):
        buf[...] = jnp.broadcast_to(emb_ref[...][None], buf.shape)
        pending = []
        for i in range(n_dma):
            if len(pending) == WINDOW:
                pending.pop(0).wait()
            cp = pltpu.make_async_copy(buf, out_ref.at[pl.ds(i * KK, KK)], sem)
            cp.start()
            pending.append(cp)
        for cp in pending:
            cp.wait()

    out = pl.pallas_call(
        body,
        in_specs=[pl.BlockSpec(memory_space=pltpu.MemorySpace.VMEM)],
        out_specs=pl.BlockSpec(memory_space=pl.ANY),
        out_shape=jax.ShapeDtypeStruct((B, Lk, E), jnp.float32),
        scratch_shapes=[
            pltpu.VMEM((KK, Lk, E), jnp.float32),
            pltpu.SemaphoreType.DMA,
        ],
    )(pos_emb)
    return out
